# baseline (speedup 1.0000x reference)
import functools

import jax
import jax.numpy as jnp
from jax import lax
from jax.experimental import pallas as pl
from jax.experimental.pallas import tpu as pltpu
from jax.experimental.pallas import tpu_sc as plsc

N_NODES = 10000
N_EDGES = 320000
D_FEAT = 128
HIDDEN = 16
N_CLASSES = 40
CP = 48

NW = 32
CH = 128
NCH = 80
E_PAD = NW * NCH * CH
K = 10
NGRP = NCH // K
NP = 10240
RPT = NP // 16
HROWS = NP // 16

_mesh = plsc.VectorSubcoreMesh(core_axis_name="c", subcore_axis_name="s")
_sc_params = pltpu.CompilerParams(
    needs_layout_passes=False, use_tc_tiling_on_sc=False
)


EPT = NP // 16


@functools.partial(
    pl.kernel,
    out_type=jax.ShapeDtypeStruct((2, NP), jnp.float32),
    mesh=_mesh,
    scratch_types=[
        pltpu.VMEM((NCH, CH), jnp.int32),
        pltpu.VMEM((NP,), jnp.float32),
        pltpu.VMEM((EPT,), jnp.float32),
        pltpu.VMEM((EPT,), jnp.float32),
        pltpu.VMEM_SHARED((16, NP), jnp.float32),
        pltpu.SemaphoreType.DMA,
    ],
    compiler_params=_sc_params,
)
def _degree_kernel(dst_hbm, out_hbm, dst_v, hist_v, tmp_v, accs_v, sh_all, sem):
    cid = lax.axis_index("c")
    sid = lax.axis_index("s")
    wid = cid * 16 + sid
    pltpu.sync_copy(dst_hbm.at[wid], dst_v)

    z = jnp.zeros((16,), jnp.float32)

    def _zero_hist(r, carry):
        hist_v[pl.ds(r * 16, 16)] = z
        return carry

    lax.fori_loop(0, NP // 16, _zero_hist, 0)

    ones = jnp.ones((16,), jnp.float32)

    def _hist_chunk(j, carry):
        for v in range(CH // 16):
            idx = dst_v[j, pl.ds(v * 16, 16)]
            plsc.addupdate_scatter(hist_v, [idx], ones)
        return carry

    lax.fori_loop(0, NCH, _hist_chunk, 0)

    pltpu.sync_copy(hist_v, sh_all.at[sid])
    plsc.subcore_barrier()

    def _zero_acc(r, carry):
        accs_v[pl.ds(r * 16, 16)] = z
        return carry

    lax.fori_loop(0, EPT // 16, _zero_acc, 0)

    for t in range(16):
        pltpu.sync_copy(sh_all.at[t, pl.ds(sid * EPT, EPT)], tmp_v)

        def _acc(r, carry):
            s = pl.ds(r * 16, 16)
            accs_v[s] = accs_v[s] + tmp_v[s]
            return carry

        lax.fori_loop(0, EPT // 16, _acc, 0)

    pltpu.sync_copy(accs_v, out_hbm.at[cid, pl.ds(sid * EPT, EPT)])


def _make_agg(width):
    @functools.partial(
        pl.kernel,
        out_type=jax.ShapeDtypeStruct((2, NP, width), jnp.float32),
        mesh=_mesh,
        scratch_types=[
            pltpu.VMEM((NCH, CH), jnp.int32),
            pltpu.VMEM((NCH, CH), jnp.int32),
            pltpu.VMEM((K, CH, width), jnp.float32),
            pltpu.VMEM_SHARED((NP, width), jnp.float32),
            pltpu.SemaphoreType.DMA,
            pltpu.SemaphoreType.DMA,
            pltpu.SemaphoreType.DMA,
        ],
        compiler_params=_sc_params,
    )
    def _agg(g_hbm, src_hbm, dst_hbm, out_hbm, src_v, dst_v, rows_v,
             acc_sh, gsem, ssem, lsem):
        cid = lax.axis_index("c")
        sid = lax.axis_index("s")
        wid = cid * 16 + sid
        ld_s = pltpu.async_copy(src_hbm.at[wid], src_v, lsem)
        ld_d = pltpu.async_copy(dst_hbm.at[wid], dst_v, lsem)

        z = jnp.zeros((16,), jnp.float32)

        def _zero(r, carry):
            for q in range(width // 16):
                rows_v[0, r, pl.ds(q * 16, 16)] = z
            return carry

        lax.fori_loop(0, CH, _zero, 0)
        nzc = RPT // CH
        zds = [
            pltpu.async_copy(
                rows_v.at[0], acc_sh.at[pl.ds(sid * RPT + b * CH, CH)], gsem
            )
            for b in range(nzc)
        ]
        for d in zds:
            d.wait()
        ld_s.wait()
        ld_d.wait()
        plsc.subcore_barrier()

        def _group(oo, carry):
            base = oo * K
            gds = [
                pltpu.async_copy(g_hbm.at[src_v.at[base + b]], rows_v.at[b], gsem)
                for b in range(K)
            ]
            for d in gds:
                d.wait()
            sds = [
                pltpu.async_copy(
                    rows_v.at[b], acc_sh.at[dst_v.at[base + b]], ssem, add=True
                )
                for b in range(K)
            ]
            for d in sds:
                d.wait()
            return carry

        lax.fori_loop(0, NGRP, _group, 0)
        plsc.subcore_barrier()

        nzc = RPT // CH
        ods = [
            pltpu.async_copy(
                acc_sh.at[pl.ds(sid * RPT + b * CH, CH)], rows_v.at[b], gsem
            )
            for b in range(nzc)
        ]
        for d in ods:
            d.wait()
        wds = [
            pltpu.async_copy(
                rows_v.at[b], out_hbm.at[cid, pl.ds(sid * RPT + b * CH, CH)], ssem
            )
            for b in range(nzc)
        ]
        for d in wds:
            d.wait()

    return _agg


_agg16 = _make_agg(HIDDEN)
_agg48 = _make_agg(CP)

_TCB = 1024


def _tc1(xp, W1, d0, d1):
    def body(x_ref, w_ref, d0_ref, d1_ref, g_ref, dinv_ref):
        deg = d0_ref[...] + d1_ref[...] + 1.0
        dinv = lax.rsqrt(deg)
        h = jnp.dot(x_ref[...], w_ref[...], preferred_element_type=jnp.float32)
        g_ref[...] = h * dinv
        dinv_ref[...] = dinv

    return pl.pallas_call(
        body,
        grid=(NP // _TCB,),
        in_specs=[
            pl.BlockSpec((_TCB, D_FEAT), lambda i: (i, 0)),
            pl.BlockSpec((D_FEAT, HIDDEN), lambda i: (0, 0)),
            pl.BlockSpec((_TCB, 1), lambda i: (i, 0)),
            pl.BlockSpec((_TCB, 1), lambda i: (i, 0)),
        ],
        out_specs=[
            pl.BlockSpec((_TCB, HIDDEN), lambda i: (i, 0)),
            pl.BlockSpec((_TCB, 1), lambda i: (i, 0)),
        ],
        out_shape=[
            jax.ShapeDtypeStruct((NP, HIDDEN), jnp.float32),
            jax.ShapeDtypeStruct((NP, 1), jnp.float32),
        ],
    )(xp, W1, d0, d1)


def _tc2(a0, a1, g1, dinv, b1, w2p):
    def body(a0_ref, a1_ref, g1_ref, dinv_ref, b1_ref, w2_ref, g2_ref):
        s1 = jnp.maximum(
            dinv_ref[...] * (a0_ref[...] + a1_ref[...] + g1_ref[...]) + b1_ref[...],
            0.0,
        )
        g2_ref[...] = dinv_ref[...] * jnp.dot(
            s1, w2_ref[...], preferred_element_type=jnp.float32
        )

    return pl.pallas_call(
        body,
        grid=(NP // _TCB,),
        in_specs=[
            pl.BlockSpec((_TCB, HIDDEN), lambda i: (i, 0)),
            pl.BlockSpec((_TCB, HIDDEN), lambda i: (i, 0)),
            pl.BlockSpec((_TCB, HIDDEN), lambda i: (i, 0)),
            pl.BlockSpec((_TCB, 1), lambda i: (i, 0)),
            pl.BlockSpec((1, HIDDEN), lambda i: (0, 0)),
            pl.BlockSpec((HIDDEN, CP), lambda i: (0, 0)),
        ],
        out_specs=pl.BlockSpec((_TCB, CP), lambda i: (i, 0)),
        out_shape=jax.ShapeDtypeStruct((NP, CP), jnp.float32),
    )(a0, a1, g1, dinv, b1, w2p)


def _tc3(a0, a1, g2, dinv, b2p):
    def body(a0_ref, a1_ref, g2_ref, dinv_ref, b2_ref, out_ref):
        out_ref[...] = (
            dinv_ref[...] * (a0_ref[...] + a1_ref[...] + g2_ref[...]) + b2_ref[...]
        )

    return pl.pallas_call(
        body,
        grid=(NP // _TCB,),
        in_specs=[
            pl.BlockSpec((_TCB, CP), lambda i: (i, 0)),
            pl.BlockSpec((_TCB, CP), lambda i: (i, 0)),
            pl.BlockSpec((_TCB, CP), lambda i: (i, 0)),
            pl.BlockSpec((_TCB, 1), lambda i: (i, 0)),
            pl.BlockSpec((1, CP), lambda i: (0, 0)),
        ],
        out_specs=pl.BlockSpec((_TCB, CP), lambda i: (i, 0)),
        out_shape=jax.ShapeDtypeStruct((NP, CP), jnp.float32),
    )(a0, a1, g2, dinv, b2p)


def kernel(x, edge_index, W1, b1, W2, b2):
    pad = E_PAD - N_EDGES
    src3 = jnp.concatenate(
        [edge_index[0], jnp.zeros((pad,), edge_index.dtype)]
    ).reshape(NW, NCH, CH)
    dst3 = jnp.concatenate(
        [edge_index[1], jnp.full((pad,), NP - 1, edge_index.dtype)]
    ).reshape(NW, NCH, CH)

    degp = _degree_kernel(dst3).reshape(2, NP, 1)

    xp = jnp.pad(x, ((0, NP - N_NODES), (0, 0)))
    g1, dinv = _tc1(xp, W1, degp[0], degp[1])

    acc1 = _agg16(g1, src3, dst3)

    b1r = b1.reshape(1, HIDDEN)
    w2p = jnp.pad(W2, ((0, 0), (0, CP - N_CLASSES)))
    g2 = _tc2(acc1[0], acc1[1], g1, dinv, b1r, w2p)

    acc2 = _agg48(g2, src3, dst3)

    b2p = jnp.pad(b2, (0, CP - N_CLASSES)).reshape(1, CP)
    out = _tc3(acc2[0], acc2[1], g2, dinv, b2p)
    return out[:N_NODES, :N_CLASSES]


# speedup vs baseline: 2.0321x; 2.0321x over previous
import functools

import jax
import jax.numpy as jnp
from jax import lax
from jax.experimental import pallas as pl
from jax.experimental.pallas import tpu as pltpu
from jax.experimental.pallas import tpu_sc as plsc

N_NODES = 10000
N_EDGES = 320000
D_FEAT = 128
HIDDEN = 16
N_CLASSES = 40
CP = 48

NW = 32
EPW = N_EDGES // NW
CH = 80
NCH = EPW // CH
K = 10
NGRP = NCH // K
TAIL = NCH - NGRP * K
NP = 10240
RPT = NP // 16
NZC = RPT // CH

_mesh = plsc.VectorSubcoreMesh(core_axis_name="c", subcore_axis_name="s")
_sc_params = pltpu.CompilerParams(
    needs_layout_passes=False, use_tc_tiling_on_sc=False
)


@functools.partial(
    pl.kernel,
    out_type=[
        jax.ShapeDtypeStruct((NP,), jnp.float32),
        jax.ShapeDtypeStruct((NP,), jnp.float32),
    ],
    mesh=_mesh,
    scratch_types=[
        pltpu.VMEM((EPW,), jnp.int32),
        pltpu.VMEM((NP,), jnp.float32),
        pltpu.VMEM((4, RPT), jnp.float32),
        pltpu.VMEM((RPT,), jnp.float32),
        pltpu.VMEM_SHARED((16, NP), jnp.float32),
        pltpu.SemaphoreType.DMA,
    ],
    compiler_params=_sc_params,
)
def _degree_kernel(edge_hbm, out0, out1, dst_v, hist_v, tmp_v, accs_v, sh_all, sem):
    cid = lax.axis_index("c")
    sid = lax.axis_index("s")
    wid = cid * 16 + sid
    ld = pltpu.async_copy(edge_hbm.at[1, pl.ds(wid * EPW, EPW)], dst_v, sem)

    z = jnp.zeros((16,), jnp.float32)

    def _zero_hist(r, carry):
        hist_v[pl.ds(r * 16, 16)] = z
        return carry

    lax.fori_loop(0, NP // 16, _zero_hist, 0)
    ld.wait()

    ones = jnp.ones((16,), jnp.float32)

    def _hist(j, carry):
        idx = dst_v[pl.ds(j * 16, 16)]
        plsc.addupdate_scatter(hist_v, [idx], ones)
        return carry

    lax.fori_loop(0, EPW // 16, _hist, 0)

    pltpu.sync_copy(hist_v, sh_all.at[sid])
    plsc.subcore_barrier()

    def _zero_acc(r, carry):
        accs_v[pl.ds(r * 16, 16)] = z
        return carry

    lax.fori_loop(0, RPT // 16, _zero_acc, 0)

    for tg in range(4):
        lds = [
            pltpu.async_copy(
                sh_all.at[tg * 4 + t, pl.ds(sid * RPT, RPT)], tmp_v.at[t], sem
            )
            for t in range(4)
        ]
        for d in lds:
            d.wait()

        def _acc(r, carry):
            s = pl.ds(r * 16, 16)
            accs_v[s] = accs_v[s] + tmp_v[0, s] + tmp_v[1, s] + tmp_v[2, s] + tmp_v[3, s]
            return carry

        lax.fori_loop(0, RPT // 16, _acc, 0)

    @pl.when(cid == 0)
    def _():
        pltpu.sync_copy(accs_v, out0.at[pl.ds(sid * RPT, RPT)])

    @pl.when(cid == 1)
    def _():
        pltpu.sync_copy(accs_v, out1.at[pl.ds(sid * RPT, RPT)])


def _make_agg(width):
    @functools.partial(
        pl.kernel,
        out_type=[
            jax.ShapeDtypeStruct((NP, width), jnp.float32),
            jax.ShapeDtypeStruct((NP, width), jnp.float32),
        ],
        mesh=_mesh,
        scratch_types=[
            pltpu.VMEM((NCH, CH), jnp.int32),
            pltpu.VMEM((NCH, CH), jnp.int32),
            pltpu.VMEM((K, CH, width), jnp.float32),
            pltpu.VMEM_SHARED((NP, width), jnp.float32),
            pltpu.SemaphoreType.DMA,
            pltpu.SemaphoreType.DMA,
            pltpu.SemaphoreType.DMA,
        ],
        compiler_params=_sc_params,
    )
    def _agg(g_hbm, edge_hbm, out0, out1, src_v, dst_v, rows_v,
             acc_sh, gsem, ssem, lsem):
        cid = lax.axis_index("c")
        sid = lax.axis_index("s")
        wid = cid * 16 + sid
        ld_s = pltpu.async_copy(edge_hbm.at[0, wid], src_v, lsem)
        ld_d = pltpu.async_copy(edge_hbm.at[1, wid], dst_v, lsem)

        z = jnp.zeros((16,), jnp.float32)

        def _zero(r, carry):
            for q in range(width // 16):
                rows_v[0, r, pl.ds(q * 16, 16)] = z
            return carry

        lax.fori_loop(0, CH, _zero, 0)
        zds = [
            pltpu.async_copy(
                rows_v.at[0], acc_sh.at[pl.ds(sid * RPT + b * CH, CH)], gsem
            )
            for b in range(NZC)
        ]
        for d in zds:
            d.wait()
        ld_s.wait()
        ld_d.wait()
        plsc.subcore_barrier()

        def _run_group(base, nk):
            gds = [
                pltpu.async_copy(g_hbm.at[src_v.at[base + b]], rows_v.at[b], gsem)
                for b in range(nk)
            ]
            for d in gds:
                d.wait()
            sds = [
                pltpu.async_copy(
                    rows_v.at[b], acc_sh.at[dst_v.at[base + b]], ssem, add=True
                )
                for b in range(nk)
            ]
            for d in sds:
                d.wait()

        def _group(oo, carry):
            _run_group(oo * K, K)
            return carry

        lax.fori_loop(0, NGRP, _group, 0)
        _run_group(NGRP * K, TAIL)
        plsc.subcore_barrier()

        ods = [
            pltpu.async_copy(
                acc_sh.at[pl.ds(sid * RPT + b * CH, CH)], rows_v.at[b], gsem
            )
            for b in range(NZC)
        ]
        for d in ods:
            d.wait()

        @pl.when(cid == 0)
        def _():
            wds = [
                pltpu.async_copy(
                    rows_v.at[b], out0.at[pl.ds(sid * RPT + b * CH, CH)], ssem
                )
                for b in range(NZC)
            ]
            for d in wds:
                d.wait()

        @pl.when(cid == 1)
        def _():
            wds = [
                pltpu.async_copy(
                    rows_v.at[b], out1.at[pl.ds(sid * RPT + b * CH, CH)], ssem
                )
                for b in range(NZC)
            ]
            for d in wds:
                d.wait()

    return _agg


_agg16 = _make_agg(HIDDEN)
_agg48 = _make_agg(CP)

_TCB = 1024
_TCB3 = 1000


def _tc1(x, W1, d0, d1):
    def body(x_ref, w_ref, d0_ref, d1_ref, g_ref, dinv_ref):
        deg = d0_ref[...] + d1_ref[...] + 1.0
        dinv = lax.rsqrt(deg)
        h = jnp.dot(x_ref[...], w_ref[...], preferred_element_type=jnp.float32)
        g_ref[...] = h * dinv
        dinv_ref[...] = dinv

    return pl.pallas_call(
        body,
        grid=(NP // _TCB,),
        in_specs=[
            pl.BlockSpec((_TCB, D_FEAT), lambda i: (i, 0)),
            pl.BlockSpec((D_FEAT, HIDDEN), lambda i: (0, 0)),
            pl.BlockSpec((_TCB, 1), lambda i: (i, 0)),
            pl.BlockSpec((_TCB, 1), lambda i: (i, 0)),
        ],
        out_specs=[
            pl.BlockSpec((_TCB, HIDDEN), lambda i: (i, 0)),
            pl.BlockSpec((_TCB, 1), lambda i: (i, 0)),
        ],
        out_shape=[
            jax.ShapeDtypeStruct((NP, HIDDEN), jnp.float32),
            jax.ShapeDtypeStruct((NP, 1), jnp.float32),
        ],
    )(x, W1, d0, d1)


def _tc2(a0, a1, g1, dinv, b1, w2p):
    def body(a0_ref, a1_ref, g1_ref, dinv_ref, b1_ref, w2_ref, g2_ref):
        s1 = jnp.maximum(
            dinv_ref[...] * (a0_ref[...] + a1_ref[...] + g1_ref[...]) + b1_ref[...],
            0.0,
        )
        g2_ref[...] = dinv_ref[...] * jnp.dot(
            s1, w2_ref[...], preferred_element_type=jnp.float32
        )

    return pl.pallas_call(
        body,
        grid=(NP // _TCB,),
        in_specs=[
            pl.BlockSpec((_TCB, HIDDEN), lambda i: (i, 0)),
            pl.BlockSpec((_TCB, HIDDEN), lambda i: (i, 0)),
            pl.BlockSpec((_TCB, HIDDEN), lambda i: (i, 0)),
            pl.BlockSpec((_TCB, 1), lambda i: (i, 0)),
            pl.BlockSpec((1, HIDDEN), lambda i: (0, 0)),
            pl.BlockSpec((HIDDEN, CP), lambda i: (0, 0)),
        ],
        out_specs=pl.BlockSpec((_TCB, CP), lambda i: (i, 0)),
        out_shape=jax.ShapeDtypeStruct((NP, CP), jnp.float32),
    )(a0, a1, g1, dinv, b1, w2p)


def _tc3(a0, a1, g2, dinv, b2):
    def body(a0_ref, a1_ref, g2_ref, dinv_ref, b2_ref, out_ref):
        s = dinv_ref[...] * (a0_ref[...] + a1_ref[...] + g2_ref[...])
        out_ref[...] = s[:, :N_CLASSES] + b2_ref[...]

    return pl.pallas_call(
        body,
        grid=(N_NODES // _TCB3,),
        in_specs=[
            pl.BlockSpec((_TCB3, CP), lambda i: (i, 0)),
            pl.BlockSpec((_TCB3, CP), lambda i: (i, 0)),
            pl.BlockSpec((_TCB3, CP), lambda i: (i, 0)),
            pl.BlockSpec((_TCB3, 1), lambda i: (i, 0)),
            pl.BlockSpec((1, N_CLASSES), lambda i: (0, 0)),
        ],
        out_specs=pl.BlockSpec((_TCB3, N_CLASSES), lambda i: (i, 0)),
        out_shape=jax.ShapeDtypeStruct((N_NODES, N_CLASSES), jnp.float32),
    )(a0, a1, g2, dinv, b2)


def kernel(x, edge_index, W1, b1, W2, b2):
    edge3 = edge_index.reshape(2, NW, NCH, CH)

    d0, d1 = _degree_kernel(edge_index)
    d0 = d0.reshape(NP, 1)
    d1 = d1.reshape(NP, 1)

    g1, dinv = _tc1(x, W1, d0, d1)

    a0, a1 = _agg16(g1, edge3)

    b1r = b1.reshape(1, HIDDEN)
    w2p = jnp.pad(W2, ((0, 0), (0, CP - N_CLASSES)))
    g2 = _tc2(a0, a1, g1, dinv, b1r, w2p)

    c0, c1 = _agg48(g2, edge3)

    b2r = b2.reshape(1, N_CLASSES)
    return _tc3(c0, c1, g2, dinv, b2r)
